# Initial kernel scaffold; baseline (speedup 1.0000x reference)
import functools

import jax
import jax.numpy as jnp
from jax import lax
from jax.experimental import pallas as pl
from jax.experimental.pallas import tpu as pltpu
from jax.experimental.pallas import tpu_sc as plsc

D_MODEL = 64
NUM_CORES = 2
NUM_SUBCORES = 16
NUM_WORKERS = NUM_CORES * NUM_SUBCORES
LANES = 16
K = 512


def _make_kernel(n_flat):
    assert n_flat % (NUM_WORKERS * K) == 0
    per_worker = n_flat // NUM_WORKERS
    n_chunks = per_worker // K
    mesh = plsc.VectorSubcoreMesh(core_axis_name="c", subcore_axis_name="s")

    @functools.partial(
        pl.kernel,
        out_type=jax.ShapeDtypeStruct((n_flat, D_MODEL), jnp.float32),
        mesh=mesh,
        scratch_types=[
            pltpu.VMEM((K,), jnp.int32),
            pltpu.VMEM((K,), jnp.int32),
            pltpu.VMEM((K, D_MODEL), jnp.float32),
            pltpu.VMEM((K, D_MODEL), jnp.float32),
            pltpu.SemaphoreType.DMA,
            pltpu.SemaphoreType.DMA,
        ],
    )
    def body(px_hbm, py_hbm, ex_hbm, ey_hbm, out_hbm,
             idxx, idxy, bufx, bufy, semx, semy):
        wid = lax.axis_index("s") * NUM_CORES + lax.axis_index("c")
        base = wid * per_worker

        def chunk_body(g, carry):
            off = base + g * K
            pltpu.sync_copy(px_hbm.at[pl.ds(off, K)], idxx)
            pltpu.sync_copy(py_hbm.at[pl.ds(off, K)], idxy)
            cx = pltpu.async_copy(ex_hbm.at[idxx], bufx, semx)
            cy = pltpu.async_copy(ey_hbm.at[idxy], bufy, semy)
            cx.wait()
            cy.wait()

            def row_body(j, carry2):
                for c in range(D_MODEL // LANES):
                    sl = pl.ds(c * LANES, LANES)
                    bufx[j, sl] = bufx[j, sl] + bufy[j, sl]
                return carry2

            lax.fori_loop(0, K, row_body, 0, unroll=2)
            pltpu.sync_copy(bufx, out_hbm.at[pl.ds(off, K)])
            return carry

        lax.fori_loop(0, n_chunks, chunk_body, 0)

    return body


def kernel(pos_x, pos_y, ex_weight, ey_weight):
    b, s = pos_x.shape
    n_flat = b * s
    px = pos_x.reshape(n_flat).astype(jnp.int32)
    py = pos_y.reshape(n_flat).astype(jnp.int32)
    out = _make_kernel(n_flat)(px, py, ex_weight, ey_weight)
    return out.reshape(b, s, D_MODEL)


# speedup vs baseline: 4.3953x; 4.3953x over previous
import functools

import jax
import jax.numpy as jnp
from jax import lax
from jax.experimental import pallas as pl
from jax.experimental.pallas import tpu as pltpu
from jax.experimental.pallas import tpu_sc as plsc

D_MODEL = 64
NUM_CORES = 2
NUM_SUBCORES = 16
NUM_WORKERS = NUM_CORES * NUM_SUBCORES
LANES = 16
K = 512


def _make_kernel(n_flat):
    assert n_flat % (NUM_WORKERS * K) == 0
    per_worker = n_flat // NUM_WORKERS
    n_chunks = per_worker // K
    mesh = plsc.VectorSubcoreMesh(core_axis_name="c", subcore_axis_name="s")

    @functools.partial(
        pl.kernel,
        out_type=jax.ShapeDtypeStruct((n_flat, D_MODEL), jnp.float32),
        mesh=mesh,
        scratch_types=[
            pltpu.VMEM((K,), jnp.int32),
            pltpu.VMEM((K,), jnp.int32),
            pltpu.VMEM((K, D_MODEL), jnp.float32),
            pltpu.VMEM((K, D_MODEL), jnp.float32),
            pltpu.SemaphoreType.DMA,
            pltpu.SemaphoreType.DMA,
        ],
        compiler_params=pltpu.CompilerParams(use_tc_tiling_on_sc=False),
    )
    def body(px_hbm, py_hbm, ex_hbm, ey_hbm, out_hbm,
             idxx, idxy, bufx, bufy, semx, semy):
        wid = lax.axis_index("s") * NUM_CORES + lax.axis_index("c")
        base = wid * per_worker

        def chunk_body(g, carry):
            off = base + g * K
            pltpu.sync_copy(px_hbm.at[pl.ds(off, K)], idxx)
            pltpu.sync_copy(py_hbm.at[pl.ds(off, K)], idxy)
            cx = pltpu.async_copy(ex_hbm.at[idxx], bufx, semx)
            cy = pltpu.async_copy(ey_hbm.at[idxy], bufy, semy)
            cx.wait()
            cy.wait()

            def row_body(j, carry2):
                for c in range(D_MODEL // LANES):
                    sl = pl.ds(c * LANES, LANES)
                    bufx[j, sl] = bufx[j, sl] + bufy[j, sl]
                return carry2

            lax.fori_loop(0, K, row_body, 0, unroll=2)
            pltpu.sync_copy(bufx, out_hbm.at[pl.ds(off, K)])
            return carry

        lax.fori_loop(0, n_chunks, chunk_body, 0)

    return body


def kernel(pos_x, pos_y, ex_weight, ey_weight):
    b, s = pos_x.shape
    n_flat = b * s
    px = pos_x.reshape(n_flat).astype(jnp.int32)
    py = pos_y.reshape(n_flat).astype(jnp.int32)
    out = _make_kernel(n_flat)(px, py, ex_weight, ey_weight)
    return out.reshape(b, s, D_MODEL)


# speedup vs baseline: 5.3791x; 1.2238x over previous
import functools

import jax
import jax.numpy as jnp
from jax import lax
from jax.experimental import pallas as pl
from jax.experimental.pallas import tpu as pltpu
from jax.experimental.pallas import tpu_sc as plsc

D_MODEL = 64
NUM_CORES = 2
NUM_SUBCORES = 16
NUM_WORKERS = NUM_CORES * NUM_SUBCORES
LANES = 16
K = 256


def _make_kernel(n_flat):
    assert n_flat % (NUM_WORKERS * K) == 0
    per_worker = n_flat // NUM_WORKERS
    n_chunks = per_worker // K
    mesh = plsc.VectorSubcoreMesh(core_axis_name="c", subcore_axis_name="s")

    @functools.partial(
        pl.kernel,
        out_type=jax.ShapeDtypeStruct((n_flat, D_MODEL), jnp.float32),
        mesh=mesh,
        scratch_types=[
            pltpu.VMEM((per_worker,), jnp.int32),
            pltpu.VMEM((per_worker,), jnp.int32),
            pltpu.VMEM((K, D_MODEL), jnp.float32),
            pltpu.VMEM((K, D_MODEL), jnp.float32),
            pltpu.VMEM((K, D_MODEL), jnp.float32),
            pltpu.VMEM((K, D_MODEL), jnp.float32),
            pltpu.SemaphoreType.DMA,
            pltpu.SemaphoreType.DMA,
            pltpu.SemaphoreType.DMA,
            pltpu.SemaphoreType.DMA,
        ],
        compiler_params=pltpu.CompilerParams(use_tc_tiling_on_sc=False),
    )
    def body(px_hbm, py_hbm, ex_hbm, ey_hbm, out_hbm,
             idxx, idxy, bufx0, bufy0, bufx1, bufy1,
             semg0, semg1, semo0, semo1):
        wid = lax.axis_index("s") * NUM_CORES + lax.axis_index("c")
        base = wid * per_worker

        pltpu.sync_copy(px_hbm.at[pl.ds(base, per_worker)], idxx)
        pltpu.sync_copy(py_hbm.at[pl.ds(base, per_worker)], idxy)

        bufs = ((bufx0, bufy0, semg0, semo0), (bufx1, bufy1, semg1, semo1))

        def issue_gathers(g, slot):
            bx, by, sg, _ = bufs[slot]
            cx = pltpu.async_copy(ex_hbm.at[idxx.at[pl.ds(g * K, K)]], bx, sg)
            cy = pltpu.async_copy(ey_hbm.at[idxy.at[pl.ds(g * K, K)]], by, sg)
            return cx, cy

        def wait_gathers(slot):
            bx, by, sg, _ = bufs[slot]
            pltpu.make_async_copy(ex_hbm.at[idxx.at[pl.ds(0, K)]], bx, sg).wait()
            pltpu.make_async_copy(ey_hbm.at[idxy.at[pl.ds(0, K)]], by, sg).wait()

        def add_and_writeout(g, slot):
            bx, by, _, so = bufs[slot]

            def row_body(j, carry):
                for c in range(D_MODEL // LANES):
                    sl = pl.ds(c * LANES, LANES)
                    bx[j, sl] = bx[j, sl] + by[j, sl]
                return carry

            lax.fori_loop(0, K, row_body, 0, unroll=4)
            pltpu.async_copy(bx, out_hbm.at[pl.ds(base + g * K, K)], so)

        def wait_out(slot):
            bx, _, _, so = bufs[slot]
            pltpu.make_async_copy(bx, out_hbm.at[pl.ds(0, K)], so).wait()

        issue_gathers(0, 0)

        def loop_body(gg, carry):
            g0 = gg * 2
            g1 = g0 + 1

            @pl.when(gg > 0)
            def _():
                wait_out(1)

            issue_gathers(g1, 1)
            wait_gathers(0)
            add_and_writeout(g0, 0)

            wait_out(0)

            @pl.when(g1 + 1 < n_chunks)
            def _():
                issue_gathers(g1 + 1, 0)

            wait_gathers(1)
            add_and_writeout(g1, 1)
            return carry

        lax.fori_loop(0, n_chunks // 2, loop_body, 0)
        wait_out(1)

    return body


def kernel(pos_x, pos_y, ex_weight, ey_weight):
    b, s = pos_x.shape
    n_flat = b * s
    px = pos_x.reshape(n_flat).astype(jnp.int32)
    py = pos_y.reshape(n_flat).astype(jnp.int32)
    out = _make_kernel(n_flat)(px, py, ex_weight, ey_weight)
    return out.reshape(b, s, D_MODEL)


# speedup vs baseline: 5.5787x; 1.0371x over previous
import functools

import jax
import jax.numpy as jnp
from jax import lax
from jax.experimental import pallas as pl
from jax.experimental.pallas import tpu as pltpu
from jax.experimental.pallas import tpu_sc as plsc

D_MODEL = 64
NUM_ROWS_X = 1025
NUM_CORES = 2
NUM_SUBCORES = 16
NUM_WORKERS = NUM_CORES * NUM_SUBCORES
LANES = 16
K = 256


def _make_kernel(n_flat):
    assert n_flat % (NUM_WORKERS * 2 * K) == 0
    per_worker = n_flat // NUM_WORKERS
    n_chunks = per_worker // K
    mesh = plsc.VectorSubcoreMesh(core_axis_name="c", subcore_axis_name="s")

    @functools.partial(
        pl.kernel,
        out_type=jax.ShapeDtypeStruct((n_flat, D_MODEL), jnp.float32),
        mesh=mesh,
        scratch_types=[
            pltpu.VMEM((NUM_ROWS_X, D_MODEL), jnp.float32),
            pltpu.VMEM((per_worker,), jnp.int32),
            pltpu.VMEM((K,), jnp.int32),
            pltpu.VMEM((K,), jnp.int32),
            pltpu.VMEM((K, D_MODEL), jnp.float32),
            pltpu.VMEM((K, D_MODEL), jnp.float32),
            pltpu.SemaphoreType.DMA,
            pltpu.SemaphoreType.DMA,
            pltpu.SemaphoreType.DMA,
            pltpu.SemaphoreType.DMA,
        ],
        compiler_params=pltpu.CompilerParams(use_tc_tiling_on_sc=False),
    )
    def body(px_hbm, py_hbm, ex_hbm, ey_hbm, out_hbm,
             ex_vmem, pxs, pys0, pys1, bufy0, bufy1,
             semg0, semg1, semo0, semo1):
        wid = lax.axis_index("s") * NUM_CORES + lax.axis_index("c")
        base = wid * per_worker

        pltpu.sync_copy(ex_hbm, ex_vmem)
        pltpu.sync_copy(px_hbm.at[pl.ds(base, per_worker)], pxs)

        bufs = ((pys0, bufy0, semg0, semo0), (pys1, bufy1, semg1, semo1))

        def issue(g, slot):
            pys, by, sg, _ = bufs[slot]
            pltpu.sync_copy(py_hbm.at[pl.ds(base + g * K, K)], pys)
            pltpu.async_copy(ey_hbm.at[pys], by, sg)

        def wait_gather(slot):
            pys, by, sg, _ = bufs[slot]
            pltpu.make_async_copy(ey_hbm.at[pys], by, sg).wait()

        def add_and_writeout(g, slot):
            _, by, _, so = bufs[slot]

            def group_body(r, carry):
                pxv = pxs[pl.ds(g * K + r * LANES, LANES)]
                for l in range(LANES):
                    pxj = pxv[l]
                    j = r * LANES + l
                    for c in range(D_MODEL // LANES):
                        sl = pl.ds(c * LANES, LANES)
                        plsc.addupdate(by.at[j, sl], ex_vmem[pxj, sl])
                return carry

            lax.fori_loop(0, K // LANES, group_body, 0)
            pltpu.async_copy(by, out_hbm.at[pl.ds(base + g * K, K)], so)

        def wait_out(slot):
            _, by, _, so = bufs[slot]
            pltpu.make_async_copy(by, out_hbm.at[pl.ds(0, K)], so).wait()

        issue(0, 0)

        def loop_body(gg, carry):
            g0 = gg * 2
            g1 = g0 + 1

            @pl.when(gg > 0)
            def _():
                wait_out(1)

            issue(g1, 1)
            wait_gather(0)
            add_and_writeout(g0, 0)

            wait_out(0)

            @pl.when(g1 + 1 < n_chunks)
            def _():
                issue(g1 + 1, 0)

            wait_gather(1)
            add_and_writeout(g1, 1)
            return carry

        lax.fori_loop(0, n_chunks // 2, loop_body, 0)
        wait_out(1)

    return body


def kernel(pos_x, pos_y, ex_weight, ey_weight):
    b, s = pos_x.shape
    n_flat = b * s
    px = pos_x.reshape(n_flat).astype(jnp.int32)
    py = pos_y.reshape(n_flat).astype(jnp.int32)
    out = _make_kernel(n_flat)(px, py, ex_weight, ey_weight)
    return out.reshape(b, s, D_MODEL)


# speedup vs baseline: 5.8782x; 1.0537x over previous
import functools

import jax
import jax.numpy as jnp
from jax import lax
from jax.experimental import pallas as pl
from jax.experimental.pallas import tpu as pltpu
from jax.experimental.pallas import tpu_sc as plsc

D_MODEL = 64
NUM_ROWS_X = 1025
NUM_CORES = 2
NUM_SUBCORES = 16
NUM_WORKERS = NUM_CORES * NUM_SUBCORES
LANES = 16


def _make_kernel(batch, seq):
    assert batch % NUM_WORKERS == 0 and seq % 8 == 0
    b_per_w = batch // NUM_WORKERS
    per_worker = b_per_w * seq
    n_full = seq // LANES
    tail = seq - n_full * LANES
    mesh = plsc.VectorSubcoreMesh(core_axis_name="c", subcore_axis_name="s")

    @functools.partial(
        pl.kernel,
        out_type=jax.ShapeDtypeStruct((batch, seq, D_MODEL), jnp.float32),
        mesh=mesh,
        scratch_types=[
            pltpu.VMEM((NUM_ROWS_X, D_MODEL), jnp.float32),
            pltpu.VMEM((per_worker,), jnp.int32),
            pltpu.VMEM((seq,), jnp.int32),
            pltpu.VMEM((seq,), jnp.int32),
            pltpu.VMEM((seq, D_MODEL), jnp.float32),
            pltpu.VMEM((seq, D_MODEL), jnp.float32),
            pltpu.SemaphoreType.DMA,
            pltpu.SemaphoreType.DMA,
            pltpu.SemaphoreType.DMA,
            pltpu.SemaphoreType.DMA,
            pltpu.SemaphoreType.DMA,
            pltpu.SemaphoreType.DMA,
        ],
        compiler_params=pltpu.CompilerParams(use_tc_tiling_on_sc=False),
    )
    def body(px_hbm, py_hbm, ex_hbm, ey_hbm, out_hbm,
             ex_vmem, pxs, pys0, pys1, buf0, buf1,
             semg0, semg1, semo0, semo1, semi0, semi1):
        wid = lax.axis_index("s") * NUM_CORES + lax.axis_index("c")
        base = wid * per_worker
        b_base = wid * b_per_w
        n_chunks = b_per_w

        pltpu.sync_copy(ex_hbm, ex_vmem)
        pltpu.sync_copy(px_hbm.at[pl.ds(base, per_worker)], pxs)

        slots = ((pys0, buf0, semg0, semo0, semi0),
                 (pys1, buf1, semg1, semo1, semi1))

        def issue_idx(g, slot):
            pys, _, _, _, si = slots[slot]
            pltpu.async_copy(py_hbm.at[pl.ds(base + g * seq, seq)], pys, si)

        def wait_idx(slot):
            pys, _, _, _, si = slots[slot]
            pltpu.make_async_copy(py_hbm.at[pl.ds(0, seq)], pys, si).wait()

        def issue_gather(slot):
            pys, buf, sg, _, _ = slots[slot]
            pltpu.async_copy(ey_hbm.at[pys], buf, sg)

        def wait_gather(slot):
            pys, buf, sg, _, _ = slots[slot]
            pltpu.make_async_copy(ey_hbm.at[pys], buf, sg).wait()

        def add_and_writeout(g, slot):
            _, buf, _, so, _ = slots[slot]
            chunk_off = g * seq

            def group_body(r, carry):
                pxv = pxs[pl.ds(chunk_off + r * LANES, LANES)]
                for l in range(LANES):
                    pxj = pxv[l]
                    j = r * LANES + l
                    for c in range(D_MODEL // LANES):
                        sl = pl.ds(c * LANES, LANES)
                        plsc.addupdate(buf.at[j, sl], ex_vmem[pxj, sl])
                return carry

            lax.fori_loop(0, n_full, group_body, 0)
            if tail:
                pxv = pxs[pl.ds(chunk_off + seq - LANES, LANES)]
                for l in range(LANES - tail, LANES):
                    pxj = pxv[l]
                    j = seq - LANES + l
                    for c in range(D_MODEL // LANES):
                        sl = pl.ds(c * LANES, LANES)
                        plsc.addupdate(buf.at[j, sl], ex_vmem[pxj, sl])
            pltpu.async_copy(buf, out_hbm.at[b_base + g], so)

        def wait_out(slot):
            _, buf, _, so, _ = slots[slot]
            pltpu.make_async_copy(buf, out_hbm.at[0], so).wait()

        issue_idx(0, 0)
        wait_idx(0)
        issue_gather(0)
        issue_idx(1, 1)

        def loop_body(gg, carry):
            g0 = gg * 2

            def phase0():
                @pl.when(gg > 0)
                def _():
                    wait_out(1)
                wait_idx(1)
                issue_gather(1)
                wait_gather(0)

                @pl.when(g0 + 2 < n_chunks)
                def _():
                    issue_idx(g0 + 2, 0)
                add_and_writeout(g0, 0)

            def phase1():
                g1 = g0 + 1
                wait_out(0)

                @pl.when(g1 + 1 < n_chunks)
                def _():
                    wait_idx(0)
                    issue_gather(0)
                wait_gather(1)

                @pl.when(g1 + 2 < n_chunks)
                def _():
                    issue_idx(g1 + 2, 1)
                add_and_writeout(g1, 1)

            phase0()
            phase1()
            return carry

        lax.fori_loop(0, n_chunks // 2, loop_body, 0)
        wait_out(1)

    return body


def kernel(pos_x, pos_y, ex_weight, ey_weight):
    b, s = pos_x.shape
    px = pos_x.reshape(b * s).astype(jnp.int32)
    py = pos_y.reshape(b * s).astype(jnp.int32)
    return _make_kernel(b, s)(px, py, ex_weight, ey_weight)
